# baseline (speedup 1.0000x reference)
import functools

import jax
import jax.numpy as jnp
from jax import lax
from jax.experimental import pallas as pl
from jax.experimental.pallas import tpu as pltpu
from jax.experimental.pallas import tpu_sc as plsc

N = 10000
E = 320000
D = 128

NC = 2
NS = 16
NW = NC * NS
C = 40
GD = 6
MAXNCH = 60
SD = 6
NP = 10240
RPT = NP // NS
_BE = 3200

SIZES = [38400, 76800, 76800, 76800, 51200]
OFFS = [0, 38400, 115200, 192000, 268800]
K = len(SIZES)

_mesh = plsc.VectorSubcoreMesh(core_axis_name="c", subcore_axis_name="s")


def _pre_body(x_ref, w1s_ref, w1d_ref, b1_ref, xs_ref, xd_ref):
    x = x_ref[...]
    xs_ref[...] = jnp.dot(x, w1s_ref[...], preferred_element_type=jnp.float32)
    xd_ref[...] = (
        jnp.dot(x, w1d_ref[...], preferred_element_type=jnp.float32) + b1_ref[...]
    )


def _precompute(x, w1s, w1d, b1):
    return pl.pallas_call(
        _pre_body,
        out_shape=(
            jax.ShapeDtypeStruct((N, D), jnp.float32),
            jax.ShapeDtypeStruct((N, D), jnp.float32),
        ),
    )(x, w1s, w1d, b1)


def _unified_scratch():
    return [
        pltpu.VMEM_SHARED((NP, D), jnp.float32),
        pltpu.VMEM((MAXNCH, C), jnp.int32),
        pltpu.VMEM((MAXNCH, C), jnp.int32),
    ] + [pltpu.VMEM((C, D), jnp.float32)] * 6 + [pltpu.SemaphoreType.DMA] * 6


def _make_gather(es, zero_acc=False):
    epw = es // NW
    nch = epw // C

    @functools.partial(
        pl.kernel,
        out_type=jax.ShapeDtypeStruct((es, D), jnp.float32),
        mesh=_mesh,
        scratch_types=_unified_scratch(),
    )
    def gather_k(xs_hbm, xd_hbm, src_hbm, dst_hbm, z_hbm, g_hbm, accres, idx_sf, idx_df, *rest):
        bufs = rest[:GD]
        sems = rest[GD:]
        cid = lax.axis_index("c")
        sid = lax.axis_index("s")
        wid = sid * NC + cid
        base = wid * epw
        pltpu.sync_copy(src_hbm.at[wid], idx_sf.at[pl.ds(0, nch)])
        pltpu.sync_copy(dst_hbm.at[wid], idx_df.at[pl.ds(0, nch)])
        if zero_acc:
            rows = pl.ds(sid * RPT, RPT)
            pltpu.sync_copy(z_hbm, accres.at[rows])
        else:
            pltpu.sync_copy(accres.at[pl.ds(0, 8)], bufs[0].at[pl.ds(0, 8)])

        L = 2
        d_xd = [None] * nch
        d_xs = [None] * nch
        d_wb = [None] * nch
        for i in range(nch + 2 * L):
            if i < nch:
                j = i
                b = j % GD
                if j >= GD:
                    d_wb[j - GD].wait()
                d_xd[j] = pltpu.async_copy(xd_hbm.at[idx_df.at[j]], bufs[b], sems[b])
            if L <= i < nch + L:
                j = i - L
                b = j % GD
                d_xd[j].wait()
                d_xs[j] = pltpu.async_copy(
                    xs_hbm.at[idx_sf.at[j]], bufs[b], sems[b], add=True
                )
            if i >= 2 * L:
                j = i - 2 * L
                b = j % GD
                d_xs[j].wait()
                off = pl.multiple_of(base + j * C, C)
                d_wb[j] = pltpu.async_copy(bufs[b], g_hbm.at[pl.ds(off, C)], sems[b])
        for j in range(max(0, nch - GD), nch):
            d_wb[j].wait()

    return gather_k


def _edge_body(
    g_ref, ea_ref, w1e_ref, w2_ref, b2_ref, wst_ref, bs_ref, eoin_ref, eo_ref, gm_ref
):
    del eoin_ref
    _edge_math(g_ref, ea_ref, w1e_ref, w2_ref, b2_ref, wst_ref, bs_ref, eo_ref, gm_ref)


def _edge_body_first(
    g_ref, ea_ref, w1e_ref, w2_ref, b2_ref, wst_ref, bs_ref, eo_ref, gm_ref
):
    _edge_math(g_ref, ea_ref, w1e_ref, w2_ref, b2_ref, wst_ref, bs_ref, eo_ref, gm_ref)


def _edge_math(g_ref, ea_ref, w1e_ref, w2_ref, b2_ref, wst_ref, bs_ref, eo_ref, gm_ref):
    ea = ea_ref[...]
    h = g_ref[...] + jnp.dot(ea, w1e_ref[...], preferred_element_type=jnp.float32)
    h = h * jax.nn.sigmoid(h)
    msg = jnp.dot(h, w2_ref[...], preferred_element_type=jnp.float32) + b2_ref[...]
    msg = msg * jax.nn.sigmoid(msg)
    eo_ref[...] = ea + msg
    ew = jax.nn.sigmoid(
        jnp.sum(msg * wst_ref[...], axis=1, keepdims=True) + bs_ref[0, 0]
    )
    gm_ref[...] = msg * ew


def _edge_mlp_slice(k, g, ea, w1e, w2, b2, wst, bs, eo_acc=None):
    es = SIZES[k]
    boff = OFFS[k] // _BE
    sblk = pl.BlockSpec((_BE, D), lambda i: (i, 0))
    fblk = pl.BlockSpec((_BE, D), lambda i, _b=boff: (i + _b, 0))
    wspec = lambda s: pl.BlockSpec(s, lambda i: tuple(0 for _ in s))
    in_specs = [
        sblk,
        fblk,
        wspec((D, D)),
        wspec((D, D)),
        wspec((1, D)),
        wspec((1, D)),
        wspec((1, 1)),
    ]
    args = [g, ea, w1e, w2, b2, wst, bs]
    aliases = {}
    body = _edge_body_first
    if eo_acc is not None:
        in_specs.append(pl.BlockSpec(memory_space=pl.ANY))
        args.append(eo_acc)
        aliases = {7: 0}
        body = _edge_body
    return pl.pallas_call(
        body,
        grid=(es // _BE,),
        in_specs=in_specs,
        out_specs=[fblk, sblk],
        out_shape=(
            jax.ShapeDtypeStruct((E, D), jnp.float32),
            jax.ShapeDtypeStruct((es, D), jnp.float32),
        ),
        input_output_aliases=aliases,
    )(*args)


def _scatter_call(k, gm, dst_r, init):
    es = SIZES[k]
    epw = es // NW
    nch = epw // C
    first = k == 0
    last = k == K - 1
    out_t = (
        jax.ShapeDtypeStruct((NC, NP, D), jnp.float32)
        if last
        else jax.ShapeDtypeStruct((8, D), jnp.float32)
    )

    @functools.partial(
        pl.kernel,
        out_type=out_t,
        mesh=_mesh,
        scratch_types=_unified_scratch(),
    )
    def scatter_k(gm_hbm, dst_hbm, init_hbm, out_hbm, acc, idx_df, idx_d2, *rest):
        del init_hbm, idx_d2
        bufs = rest[:SD]
        sems = rest[SD : 2 * SD]
        cid = lax.axis_index("c")
        sid = lax.axis_index("s")
        wid = sid * NC + cid
        base = wid * epw
        rows = pl.ds(sid * RPT, RPT)
        pltpu.sync_copy(dst_hbm.at[wid], idx_df.at[pl.ds(0, nch)])

        L = 2
        d_ld = [None] * nch
        d_sc = [None] * nch
        for i in range(nch + L):
            if i < nch:
                j = i
                b = j % SD
                if j >= SD:
                    d_sc[j - SD].wait()
                off = pl.multiple_of(base + j * C, C)
                d_ld[j] = pltpu.async_copy(gm_hbm.at[pl.ds(off, C)], bufs[b], sems[b])
            if i >= L:
                j = i - L
                b = j % SD
                d_ld[j].wait()
                d_sc[j] = pltpu.async_copy(
                    bufs[b], acc.at[idx_df.at[j]], sems[b], add=True
                )
        for j in range(max(0, nch - SD), nch):
            d_sc[j].wait()
        if last:
            plsc.subcore_barrier()
            pltpu.sync_copy(acc.at[rows], out_hbm.at[cid, rows])

    return scatter_k(gm, dst_r, init)


def _node_body(
    p_ref, x_ref, u1_ref, bu1_ref, gamma_ref, beta_ref, u2_ref, bu2_ref, out_ref
):
    x = x_ref[...]
    inp = p_ref[0, :N, :] + p_ref[1, :N, :] + x
    u = jnp.dot(inp, u1_ref[...], preferred_element_type=jnp.float32) + bu1_ref[...]
    u = u * jax.nn.sigmoid(u)
    mean = jnp.mean(u, axis=0, keepdims=True)
    var = jnp.mean((u - mean) * (u - mean), axis=0, keepdims=True)
    un = (u - mean) / jnp.sqrt(var + 1e-5) * gamma_ref[...] + beta_ref[...]
    out_ref[...] = (
        jnp.dot(un, u2_ref[...], preferred_element_type=jnp.float32) + bu2_ref[...] + x
    )


def _node_update(parts, x, u1, bu1, gamma, beta, u2, bu2):
    return pl.pallas_call(
        _node_body,
        out_shape=jax.ShapeDtypeStruct((N, D), jnp.float32),
    )(parts, x, u1, bu1, gamma, beta, u2, bu2)


def kernel(x, edge_index, edge_attr, W1, b1, W2, b2, Ws, bs, U1, bu1, gamma, beta, U2, bu2):
    src = edge_index[0]
    dst = edge_index[1]
    srcs, dsts = [], []
    for k in range(K):
        es = SIZES[k]
        nch = es // NW // C
        sl = slice(OFFS[k], OFFS[k] + es)
        srcs.append(src[sl].reshape(NW, nch, C))
        dsts.append(dst[sl].reshape(NW, nch, C))
    w1s = W1[0:D]
    w1d = W1[D : 2 * D]
    w1e = W1[2 * D :]

    xs, xd = _precompute(x, w1s, w1d, b1.reshape(1, D))

    zeros = jnp.zeros((RPT, D), jnp.float32)
    gs = [
        _make_gather(SIZES[k], zero_acc=(k == 0))(xs, xd, srcs[k], dsts[k], zeros)
        for k in range(K)
    ]

    eo_acc = None
    gms = []
    for k in range(K):
        eo_acc, gm = _edge_mlp_slice(
            k, gs[k], edge_attr, w1e, W2, b2.reshape(1, D),
            Ws.reshape(1, D), bs.reshape(1, 1), eo_acc,
        )
        gms.append(gm)

    part = x
    for k in range(K):
        part = _scatter_call(k, gms[k], dsts[k], part)

    feat = _node_update(
        part,
        x,
        U1,
        bu1.reshape(1, D),
        gamma.reshape(1, D),
        beta.reshape(1, D),
        U2,
        bu2.reshape(1, D),
    )
    return feat, eo_acc


# speedup vs baseline: 1.0298x; 1.0298x over previous
import functools

import jax
import jax.numpy as jnp
from jax import lax
from jax.experimental import pallas as pl
from jax.experimental.pallas import tpu as pltpu
from jax.experimental.pallas import tpu_sc as plsc

N = 10000
E = 320000
D = 128

NC = 2
NS = 16
NW = NC * NS
C = 80
GD = 4
MAXNCH = 30
SD = 4
NP = 10240
RPT = NP // NS
_BE = 3200

SIZES = [38400, 76800, 76800, 76800, 51200]
OFFS = [0, 38400, 115200, 192000, 268800]
K = len(SIZES)

_mesh = plsc.VectorSubcoreMesh(core_axis_name="c", subcore_axis_name="s")


def _pre_body(x_ref, w1s_ref, w1d_ref, b1_ref, xs_ref, xd_ref):
    x = x_ref[...]
    xs_ref[...] = jnp.dot(x, w1s_ref[...], preferred_element_type=jnp.float32)
    xd_ref[...] = (
        jnp.dot(x, w1d_ref[...], preferred_element_type=jnp.float32) + b1_ref[...]
    )


def _precompute(x, w1s, w1d, b1):
    return pl.pallas_call(
        _pre_body,
        out_shape=(
            jax.ShapeDtypeStruct((N, D), jnp.float32),
            jax.ShapeDtypeStruct((N, D), jnp.float32),
        ),
    )(x, w1s, w1d, b1)


def _unified_scratch():
    return [
        pltpu.VMEM_SHARED((NP, D), jnp.float32),
        pltpu.VMEM((MAXNCH, C), jnp.int32),
        pltpu.VMEM((MAXNCH, C), jnp.int32),
    ] + [pltpu.VMEM((C, D), jnp.float32)] * 4 + [pltpu.SemaphoreType.DMA] * 4


def _make_gather(es, zero_acc=False):
    epw = es // NW
    nch = epw // C

    @functools.partial(
        pl.kernel,
        out_type=jax.ShapeDtypeStruct((es, D), jnp.float32),
        mesh=_mesh,
        scratch_types=_unified_scratch(),
    )
    def gather_k(xs_hbm, xd_hbm, src_hbm, dst_hbm, z_hbm, g_hbm, accres, idx_sf, idx_df, *rest):
        bufs = rest[:GD]
        sems = rest[GD:]
        cid = lax.axis_index("c")
        sid = lax.axis_index("s")
        wid = sid * NC + cid
        base = wid * epw
        pltpu.sync_copy(src_hbm.at[wid], idx_sf.at[pl.ds(0, nch)])
        pltpu.sync_copy(dst_hbm.at[wid], idx_df.at[pl.ds(0, nch)])
        if zero_acc:
            zrows = pl.ds(lax.axis_index("s") * RPT, RPT)
            pltpu.sync_copy(z_hbm, accres.at[zrows])
        else:
            pltpu.sync_copy(accres.at[pl.ds(0, 8)], bufs[0].at[pl.ds(0, 8)])

        L = 1
        d_xd = [None] * nch
        d_xs = [None] * nch
        d_wb = [None] * nch
        for i in range(nch + 2 * L):
            if i < nch:
                j = i
                b = j % GD
                if j >= GD:
                    d_wb[j - GD].wait()
                d_xd[j] = pltpu.async_copy(xd_hbm.at[idx_df.at[j]], bufs[b], sems[b])
            if L <= i < nch + L:
                j = i - L
                b = j % GD
                d_xd[j].wait()
                d_xs[j] = pltpu.async_copy(
                    xs_hbm.at[idx_sf.at[j]], bufs[b], sems[b], add=True
                )
            if i >= 2 * L:
                j = i - 2 * L
                b = j % GD
                d_xs[j].wait()
                off = pl.multiple_of(base + j * C, C)
                d_wb[j] = pltpu.async_copy(bufs[b], g_hbm.at[pl.ds(off, C)], sems[b])
        for j in range(max(0, nch - GD), nch):
            d_wb[j].wait()

    return gather_k


def _edge_body(
    g_ref, ea_ref, w1e_ref, w2_ref, b2_ref, wst_ref, bs_ref, eoin_ref, eo_ref, gm_ref
):
    del eoin_ref
    _edge_math(g_ref, ea_ref, w1e_ref, w2_ref, b2_ref, wst_ref, bs_ref, eo_ref, gm_ref)


def _edge_body_first(
    g_ref, ea_ref, w1e_ref, w2_ref, b2_ref, wst_ref, bs_ref, eo_ref, gm_ref
):
    _edge_math(g_ref, ea_ref, w1e_ref, w2_ref, b2_ref, wst_ref, bs_ref, eo_ref, gm_ref)


def _edge_math(g_ref, ea_ref, w1e_ref, w2_ref, b2_ref, wst_ref, bs_ref, eo_ref, gm_ref):
    ea = ea_ref[...]
    h = g_ref[...] + jnp.dot(ea, w1e_ref[...], preferred_element_type=jnp.float32)
    h = h * jax.nn.sigmoid(h)
    msg = jnp.dot(h, w2_ref[...], preferred_element_type=jnp.float32) + b2_ref[...]
    msg = msg * jax.nn.sigmoid(msg)
    eo_ref[...] = ea + msg
    ew = jax.nn.sigmoid(
        jnp.sum(msg * wst_ref[...], axis=1, keepdims=True) + bs_ref[0, 0]
    )
    gm_ref[...] = msg * ew


def _edge_mlp_slice(k, g, ea, w1e, w2, b2, wst, bs, eo_acc=None):
    es = SIZES[k]
    boff = OFFS[k] // _BE
    sblk = pl.BlockSpec((_BE, D), lambda i: (i, 0))
    fblk = pl.BlockSpec((_BE, D), lambda i, _b=boff: (i + _b, 0))
    wspec = lambda s: pl.BlockSpec(s, lambda i: tuple(0 for _ in s))
    in_specs = [
        sblk,
        fblk,
        wspec((D, D)),
        wspec((D, D)),
        wspec((1, D)),
        wspec((1, D)),
        wspec((1, 1)),
    ]
    args = [g, ea, w1e, w2, b2, wst, bs]
    aliases = {}
    body = _edge_body_first
    if eo_acc is not None:
        in_specs.append(pl.BlockSpec(memory_space=pl.ANY))
        args.append(eo_acc)
        aliases = {7: 0}
        body = _edge_body
    return pl.pallas_call(
        body,
        grid=(es // _BE,),
        in_specs=in_specs,
        out_specs=[fblk, sblk],
        out_shape=(
            jax.ShapeDtypeStruct((E, D), jnp.float32),
            jax.ShapeDtypeStruct((es, D), jnp.float32),
        ),
        input_output_aliases=aliases,
    )(*args)


def _scatter_call(k, gm, dst_r, init):
    es = SIZES[k]
    epw = es // NW
    nch = epw // C
    first = k == 0
    last = k == K - 1
    out_t = (
        jax.ShapeDtypeStruct((NC, NP, D), jnp.float32)
        if last
        else jax.ShapeDtypeStruct((8, D), jnp.float32)
    )

    @functools.partial(
        pl.kernel,
        out_type=out_t,
        mesh=_mesh,
        scratch_types=_unified_scratch(),
    )
    def scatter_k(gm_hbm, dst_hbm, init_hbm, out_hbm, acc, idx_df, idx_d2, *rest):
        del init_hbm, idx_d2
        bufs = rest[:SD]
        sems = rest[SD : 2 * SD]
        cid = lax.axis_index("c")
        sid = lax.axis_index("s")
        wid = sid * NC + cid
        base = wid * epw
        rows = pl.ds(sid * RPT, RPT)
        pltpu.sync_copy(dst_hbm.at[wid], idx_df.at[pl.ds(0, nch)])

        L = 2
        d_ld = [None] * nch
        d_sc = [None] * nch
        for i in range(nch + L):
            if i < nch:
                j = i
                b = j % SD
                if j >= SD:
                    d_sc[j - SD].wait()
                off = pl.multiple_of(base + j * C, C)
                d_ld[j] = pltpu.async_copy(gm_hbm.at[pl.ds(off, C)], bufs[b], sems[b])
            if i >= L:
                j = i - L
                b = j % SD
                d_ld[j].wait()
                d_sc[j] = pltpu.async_copy(
                    bufs[b], acc.at[idx_df.at[j]], sems[b], add=True
                )
        for j in range(max(0, nch - SD), nch):
            d_sc[j].wait()
        if last:
            plsc.subcore_barrier()
            pltpu.sync_copy(acc.at[rows], out_hbm.at[cid, rows])

    return scatter_k(gm, dst_r, init)


def _node_body(
    p_ref, x_ref, u1_ref, bu1_ref, gamma_ref, beta_ref, u2_ref, bu2_ref, out_ref
):
    x = x_ref[...]
    inp = p_ref[0, :N, :] + p_ref[1, :N, :] + x
    u = jnp.dot(inp, u1_ref[...], preferred_element_type=jnp.float32) + bu1_ref[...]
    u = u * jax.nn.sigmoid(u)
    mean = jnp.mean(u, axis=0, keepdims=True)
    var = jnp.mean((u - mean) * (u - mean), axis=0, keepdims=True)
    un = (u - mean) / jnp.sqrt(var + 1e-5) * gamma_ref[...] + beta_ref[...]
    out_ref[...] = (
        jnp.dot(un, u2_ref[...], preferred_element_type=jnp.float32) + bu2_ref[...] + x
    )


def _node_update(parts, x, u1, bu1, gamma, beta, u2, bu2):
    return pl.pallas_call(
        _node_body,
        out_shape=jax.ShapeDtypeStruct((N, D), jnp.float32),
    )(parts, x, u1, bu1, gamma, beta, u2, bu2)


def kernel(x, edge_index, edge_attr, W1, b1, W2, b2, Ws, bs, U1, bu1, gamma, beta, U2, bu2):
    src = edge_index[0]
    dst = edge_index[1]
    srcs, dsts = [], []
    for k in range(K):
        es = SIZES[k]
        nch = es // NW // C
        sl = slice(OFFS[k], OFFS[k] + es)
        srcs.append(src[sl].reshape(NW, nch, C))
        dsts.append(dst[sl].reshape(NW, nch, C))
    w1s = W1[0:D]
    w1d = W1[D : 2 * D]
    w1e = W1[2 * D :]

    xs, xd = _precompute(x, w1s, w1d, b1.reshape(1, D))

    zeros = jnp.zeros((RPT, D), jnp.float32)
    gs = [
        _make_gather(SIZES[k], zero_acc=(k == 0))(xs, xd, srcs[k], dsts[k], zeros)
        for k in range(K)
    ]

    eo_acc = None
    gms = []
    for k in range(K):
        eo_acc, gm = _edge_mlp_slice(
            k, gs[k], edge_attr, w1e, W2, b2.reshape(1, D),
            Ws.reshape(1, D), bs.reshape(1, 1), eo_acc,
        )
        gms.append(gm)

    part = x
    for k in range(K):
        part = _scatter_call(k, gms[k], dsts[k], part)

    feat = _node_update(
        part,
        x,
        U1,
        bu1.reshape(1, D),
        gamma.reshape(1, D),
        beta.reshape(1, D),
        U2,
        bu2.reshape(1, D),
    )
    return feat, eo_acc
